# baseline (speedup 1.0000x reference)
import functools

import jax
import jax.numpy as jnp
from jax import lax
from jax.experimental import pallas as pl
from jax.experimental.pallas import tpu as pltpu
from jax.experimental.pallas import tpu_sc as plsc

N = 10000
E = 160000
D = 256

NC = 2
NS = 16
NW = NC * NS

ES0 = 79872
ES1 = E - ES0
SLICES = ((0, ES0), (ES0, ES1))

SC = 128
GC = 96

RPS = 624
RPS_LAST = N - RPS * (NS - 1)
CH = D // NC


def _dot(a, b):
    return jnp.dot(a, b, preferred_element_type=jnp.float32)


def _node_enc_body(x, w1, b1, w2, b2, wea, web, nd_o, a_o, b_o):
    h = jax.nn.gelu(_dot(x[...], w1[...]) + b1[...])
    nd = _dot(h, w2[...]) + b2[...]
    nd_o[...] = nd
    a_o[...] = _dot(nd, wea[...])
    b_o[...] = _dot(nd, web[...])


def _node_net_body(x, m0, m1, w1a, w1b, b1, w2, b2, wea, web,
                   nd_o, a_o, b_o):
    m = m0[...] + m1[...]
    h = jax.nn.gelu(_dot(x[...], w1a[...]) + _dot(m, w1b[...]) + b1[...])
    nd = _dot(h, w2[...]) + b2[...] + x[...]
    nd_o[...] = nd
    a_o[...] = _dot(nd, wea[...])
    b_o[...] = _dot(nd, web[...])


def _edge_enc_body(ga, gb, b1, w2, b2, e_o):
    h = jax.nn.gelu(ga[...] + gb[...] + b1[...])
    e_o[...] = _dot(h, w2[...]) + b2[...]


def _edge_net_body(ga, gb, e, wc, b1, w2, b2, e_o):
    h = jax.nn.gelu(ga[...] + gb[...] + _dot(e[...], wc[...]) + b1[...])
    e_o[...] = _dot(h, w2[...]) + b2[...] + e[...]


_RN = 1000
_RE = 256


def _bs_rows(r):
    return pl.BlockSpec((r, D), lambda i: (i, 0))


_BS_W = pl.BlockSpec((D, D), lambda i: (0, 0))
_BS_B = pl.BlockSpec((1, D), lambda i: (0, 0))


def _tc_node_encoder(x, w1, b1, w2, b2, wea, web):
    return pl.pallas_call(
        _node_enc_body,
        grid=(N // _RN,),
        in_specs=[_bs_rows(_RN), _BS_W, _BS_B, _BS_W, _BS_B, _BS_W, _BS_W],
        out_specs=[_bs_rows(_RN)] * 3,
        out_shape=[jax.ShapeDtypeStruct((N, D), jnp.float32)] * 3,
    )(x, w1, b1, w2, b2, wea, web)


def _tc_node_net(x, m0, m1, w1a, w1b, b1, w2, b2, wea, web):
    return pl.pallas_call(
        _node_net_body,
        grid=(N // _RN,),
        in_specs=[_bs_rows(_RN)] * 3 + [_BS_W, _BS_W, _BS_B, _BS_W,
                                        _BS_B, _BS_W, _BS_W],
        out_specs=[_bs_rows(_RN)] * 3,
        out_shape=[jax.ShapeDtypeStruct((N, D), jnp.float32)] * 3,
    )(x, m0, m1, w1a, w1b, b1, w2, b2, wea, web)


def _tc_edge_encoder(ga, gb, b1, w2, b2):
    es = ga.shape[0]
    return pl.pallas_call(
        _edge_enc_body,
        grid=(es // _RE,),
        in_specs=[_bs_rows(_RE), _bs_rows(_RE), _BS_B, _BS_W, _BS_B],
        out_specs=_bs_rows(_RE),
        out_shape=jax.ShapeDtypeStruct((es, D), jnp.float32),
    )(ga, gb, b1, w2, b2)


def _tc_edge_net(ga, gb, e, wc, b1, w2, b2):
    es = ga.shape[0]
    return pl.pallas_call(
        _edge_net_body,
        grid=(es // _RE,),
        in_specs=[_bs_rows(_RE), _bs_rows(_RE), _bs_rows(_RE), _BS_W, _BS_B,
                  _BS_W, _BS_B],
        out_specs=_bs_rows(_RE),
        out_shape=jax.ShapeDtypeStruct((es, D), jnp.float32),
    )(ga, gb, e, wc, b1, w2, b2)


def _make_gather2(e0, es):
    epw = es // NW
    full = epw // GC
    tail = epw - full * GC
    pairs = (full - 2) // 2
    odd = (full - 2) % 2

    def body(a_hbm, b_hbm, src_hbm, dst_hbm, ga_hbm, gb_hbm,
             idxa, idxb, ra0, rb0, ra1, rb1, sg0, sg1, sw0, sw1):
        c = lax.axis_index("c")
        s = lax.axis_index("s")
        wid = s * NC + c
        w0 = wid * epw
        wg = e0 + w0

        pltpu.sync_copy(src_hbm.at[pl.ds(wg, epw)], idxa)
        pltpu.sync_copy(dst_hbm.at[pl.ds(wg, epw)], idxb)

        def fire_g(j, ra, rb, sg):
            off = j * GC
            pltpu.async_copy(a_hbm.at[idxa.at[pl.ds(off, GC)]], ra, sg)
            pltpu.async_copy(b_hbm.at[idxb.at[pl.ds(off, GC)]], rb, sg)

        def wait_g(ra, rb, sg):
            pltpu.make_async_copy(a_hbm.at[pl.ds(0, GC)], ra, sg).wait()
            pltpu.make_async_copy(b_hbm.at[pl.ds(0, GC)], rb, sg).wait()

        def fire_w(j, ra, rb, sw):
            base = w0 + j * GC
            pltpu.async_copy(ra, ga_hbm.at[pl.ds(base, GC)], sw)
            pltpu.async_copy(rb, gb_hbm.at[pl.ds(base, GC)], sw)

        def wait_w(ra, rb, sw):
            pltpu.make_async_copy(ra, ga_hbm.at[pl.ds(0, GC)], sw).wait()
            pltpu.make_async_copy(rb, gb_hbm.at[pl.ds(0, GC)], sw).wait()

        fire_g(0, ra0, rb0, sg0)
        fire_g(1, ra1, rb1, sg1)
        wait_g(ra0, rb0, sg0)
        fire_w(0, ra0, rb0, sw0)
        wait_g(ra1, rb1, sg1)
        fire_w(1, ra1, rb1, sw1)

        def loop(t, carry):
            j0 = 2 + 2 * t
            wait_w(ra0, rb0, sw0)
            fire_g(j0, ra0, rb0, sg0)
            wait_w(ra1, rb1, sw1)
            fire_g(j0 + 1, ra1, rb1, sg1)
            wait_g(ra0, rb0, sg0)
            fire_w(j0, ra0, rb0, sw0)
            wait_g(ra1, rb1, sg1)
            fire_w(j0 + 1, ra1, rb1, sw1)
            return carry

        lax.fori_loop(0, pairs, loop, 0)
        if odd:
            wait_w(ra0, rb0, sw0)
            fire_g(full - 1, ra0, rb0, sg0)
            wait_g(ra0, rb0, sg0)
            fire_w(full - 1, ra0, rb0, sw0)
        wait_w(ra0, rb0, sw0)
        wait_w(ra1, rb1, sw1)

        if tail:
            toff = full * GC
            tbase = w0 + toff
            pltpu.async_copy(a_hbm.at[idxa.at[pl.ds(toff, tail)]],
                             ra0.at[pl.ds(0, tail)], sg0)
            pltpu.async_copy(b_hbm.at[idxb.at[pl.ds(toff, tail)]],
                             rb0.at[pl.ds(0, tail)], sg0)
            pltpu.make_async_copy(a_hbm.at[pl.ds(0, tail)],
                                  ra0.at[pl.ds(0, tail)], sg0).wait()
            pltpu.make_async_copy(b_hbm.at[pl.ds(0, tail)],
                                  rb0.at[pl.ds(0, tail)], sg0).wait()
            pltpu.sync_copy(ra0.at[pl.ds(0, tail)],
                            ga_hbm.at[pl.ds(tbase, tail)])
            pltpu.sync_copy(rb0.at[pl.ds(0, tail)],
                            gb_hbm.at[pl.ds(tbase, tail)])

    def call(a, b, src, dst):
        f = pl.kernel(
            body,
            out_type=[jax.ShapeDtypeStruct((es, D), jnp.float32)] * 2,
            mesh=plsc.VectorSubcoreMesh(core_axis_name="c",
                                        subcore_axis_name="s"),
            scratch_types=[
                pltpu.VMEM((epw,), jnp.int32),
                pltpu.VMEM((epw,), jnp.int32),
                pltpu.VMEM((GC, D), jnp.float32),
                pltpu.VMEM((GC, D), jnp.float32),
                pltpu.VMEM((GC, D), jnp.float32),
                pltpu.VMEM((GC, D), jnp.float32),
                pltpu.SemaphoreType.DMA,
                pltpu.SemaphoreType.DMA,
                pltpu.SemaphoreType.DMA,
                pltpu.SemaphoreType.DMA,
            ],
        )
        return f(a, b, src, dst)

    return call


def _make_segsum(e0, es):
    nchunks = es // SC
    n0 = nchunks // NS
    extra = nchunks - n0 * NS
    pairs = (n0 - 2) // 2
    odd = (n0 - 2) % 2

    def body(e_hbm, dst_hbm, zeros_hbm, msg_hbm,
             idx0, rows0, idx1, rows1, sl0, sl1, ss0, ss1, acc):
        c = lax.axis_index("c")
        s = lax.axis_index("s")

        @pl.when(s < NS - 1)
        def _():
            pltpu.sync_copy(zeros_hbm.at[pl.ds(0, RPS)],
                            acc.at[pl.ds(s * RPS, RPS)])

        @pl.when(s == NS - 1)
        def _():
            pltpu.sync_copy(zeros_hbm, acc.at[pl.ds((NS - 1) * RPS, RPS_LAST)])

        plsc.subcore_barrier()

        def fire_l(j, idx, rows, sl):
            base = (s + NS * j) * SC
            pltpu.async_copy(dst_hbm.at[pl.ds(e0 + base, SC)], idx, sl)
            pltpu.async_copy(e_hbm.at[pl.ds(base, SC), pl.ds(c * CH, CH)],
                             rows, sl)

        def wait_l(idx, rows, sl):
            pltpu.make_async_copy(dst_hbm.at[pl.ds(0, SC)], idx, sl).wait()
            pltpu.make_async_copy(e_hbm.at[pl.ds(0, SC), pl.ds(0, CH)],
                                  rows, sl).wait()

        def fire_s(idx, rows, ss):
            pltpu.async_copy(rows, acc.at[idx], ss, add=True)

        def wait_s(idx, rows, ss):
            pltpu.make_async_copy(rows, acc.at[idx], ss).wait()

        fire_l(0, idx0, rows0, sl0)
        fire_l(1, idx1, rows1, sl1)
        wait_l(idx0, rows0, sl0)
        fire_s(idx0, rows0, ss0)
        wait_l(idx1, rows1, sl1)
        fire_s(idx1, rows1, ss1)

        def loop(t, carry):
            j0 = 2 + 2 * t
            wait_s(idx0, rows0, ss0)
            fire_l(j0, idx0, rows0, sl0)
            wait_s(idx1, rows1, ss1)
            fire_l(j0 + 1, idx1, rows1, sl1)
            wait_l(idx0, rows0, sl0)
            fire_s(idx0, rows0, ss0)
            wait_l(idx1, rows1, sl1)
            fire_s(idx1, rows1, ss1)
            return carry

        lax.fori_loop(0, pairs, loop, 0)
        if odd:
            wait_s(idx0, rows0, ss0)
            fire_l(n0 - 1, idx0, rows0, sl0)
            wait_l(idx0, rows0, sl0)
            fire_s(idx0, rows0, ss0)
        wait_s(idx0, rows0, ss0)
        wait_s(idx1, rows1, ss1)

        if extra:
            @pl.when(s < extra)
            def _():
                base = (NS * n0 + s) * SC
                pltpu.sync_copy(dst_hbm.at[pl.ds(e0 + base, SC)], idx0)
                pltpu.sync_copy(
                    e_hbm.at[pl.ds(base, SC), pl.ds(c * CH, CH)], rows0)
                pltpu.sync_copy(rows0, acc.at[idx0], add=True)

        plsc.subcore_barrier()

        @pl.when(s < NS - 1)
        def _():
            pltpu.sync_copy(acc.at[pl.ds(s * RPS, RPS)],
                            msg_hbm.at[pl.ds(s * RPS, RPS), pl.ds(c * CH, CH)])

        @pl.when(s == NS - 1)
        def _():
            pltpu.sync_copy(
                acc.at[pl.ds((NS - 1) * RPS, RPS_LAST)],
                msg_hbm.at[pl.ds((NS - 1) * RPS, RPS_LAST), pl.ds(c * CH, CH)])

    def call(edges_slice, dst, zeros):
        f = pl.kernel(
            body,
            out_type=jax.ShapeDtypeStruct((N, D), jnp.float32),
            mesh=plsc.VectorSubcoreMesh(core_axis_name="c",
                                        subcore_axis_name="s"),
            scratch_types=[
                pltpu.VMEM((SC,), jnp.int32),
                pltpu.VMEM((SC, CH), jnp.float32),
                pltpu.VMEM((SC,), jnp.int32),
                pltpu.VMEM((SC, CH), jnp.float32),
                pltpu.SemaphoreType.DMA,
                pltpu.SemaphoreType.DMA,
                pltpu.SemaphoreType.DMA,
                pltpu.SemaphoreType.DMA,
                pltpu.VMEM_SHARED((N, CH), jnp.float32),
            ],
        )
        return f(edges_slice, dst, zeros)

    return call


_GATHER = tuple(_make_gather2(e0, es) for e0, es in SLICES)
_SEGSUM = tuple(_make_segsum(e0, es) for e0, es in SLICES)


def kernel(node_attr, graph, params):
    src = graph[0].astype(jnp.int32)
    dst = graph[1].astype(jnp.int32)

    enc = params["node_encoder"]
    ee = params["edge_encoder"]
    w1n, b1n = enc[0]["W"], enc[0]["b"].reshape(1, D)
    w2n, b2n = enc[1]["W"], enc[1]["b"].reshape(1, D)
    we1, be1 = ee[0]["W"], ee[0]["b"].reshape(1, D)
    we2, be2 = ee[1]["W"], ee[1]["b"].reshape(1, D)

    nodes, a, b = _tc_node_encoder(node_attr, w1n, b1n, w2n, b2n,
                                   we1[:D], we1[D:])
    ga0, gb0 = _GATHER[0](a, b, src, dst)
    e_0 = _tc_edge_encoder(ga0, gb0, be1, we2, be2)
    ga1, gb1 = _GATHER[1](a, b, src, dst)
    e_1 = _tc_edge_encoder(ga1, gb1, be1, we2, be2)

    zeros = jnp.zeros((RPS_LAST, CH), jnp.float32)
    for cell in params["cells"]:
        nw, ew = cell["node_network"], cell["edge_network"]
        wn1, bn1 = nw[0]["W"], nw[0]["b"].reshape(1, D)
        wn2, bn2 = nw[1]["W"], nw[1]["b"].reshape(1, D)
        wc1, bc1 = ew[0]["W"], ew[0]["b"].reshape(1, D)
        wc2, bc2 = ew[1]["W"], ew[1]["b"].reshape(1, D)

        m0 = _SEGSUM[0](e_0, dst, zeros)
        m1 = _SEGSUM[1](e_1, dst, zeros)
        nodes, a, b = _tc_node_net(nodes, m0, m1, wn1[:D], wn1[D:], bn1,
                                   wn2, bn2, wc1[:D], wc1[D:2 * D])
        ga0, gb0 = _GATHER[0](a, b, src, dst)
        e_0 = _tc_edge_net(ga0, gb0, e_0, wc1[2 * D:], bc1, wc2, bc2)
        ga1, gb1 = _GATHER[1](a, b, src, dst)
        e_1 = _tc_edge_net(ga1, gb1, e_1, wc1[2 * D:], bc1, wc2, bc2)

    return (nodes, jnp.concatenate([e_0, e_1], axis=0))


# speedup vs baseline: 1.3491x; 1.3491x over previous
import functools

import jax
import jax.numpy as jnp
import numpy as np
from jax import lax
from jax.experimental import pallas as pl
from jax.experimental.pallas import tpu as pltpu
from jax.experimental.pallas import tpu_sc as plsc

N = 10000
E = 160000
D = 256

NC = 2
NS = 16
NW = NC * NS

SC = 128
GC = 112
EPW = E // NW

RPS = 624
RPS_LAST = N - RPS * (NS - 1)
CH = D // NC


def _dot(a, b):
    return jnp.dot(a, b, preferred_element_type=jnp.float32)


def _node_enc_body(x, w1, b1, w2, b2, wea, web, nd_o, a_o, b_o):
    h = jax.nn.gelu(_dot(x[...], w1[...]) + b1[...])
    nd = _dot(h, w2[...]) + b2[...]
    nd_o[...] = nd
    a_o[...] = _dot(nd, wea[...]).astype(jnp.bfloat16)
    b_o[...] = _dot(nd, web[...]).astype(jnp.bfloat16)


def _node_net_body(x, m, w1a, w1b, b1, w2, b2, wea, web, nd_o, a_o, b_o):
    h = jax.nn.gelu(_dot(x[...], w1a[...]) + _dot(m[...], w1b[...]) + b1[...])
    nd = _dot(h, w2[...]) + b2[...] + x[...]
    nd_o[...] = nd
    a_o[...] = _dot(nd, wea[...]).astype(jnp.bfloat16)
    b_o[...] = _dot(nd, web[...]).astype(jnp.bfloat16)


_HI_MASK = np.int32(-65536)


def _unpack_sum(a, b):
    lo = (pltpu.bitcast(a << 16, jnp.float32)
          + pltpu.bitcast(b << 16, jnp.float32))
    hi = (pltpu.bitcast(a & _HI_MASK, jnp.float32)
          + pltpu.bitcast(b & _HI_MASK, jnp.float32))
    return jnp.concatenate([lo, hi], axis=1)


def _edge_enc_body(ga, gb, b1, w2, b2, e_o):
    h = jax.nn.gelu(_unpack_sum(ga[...], gb[...]) + b1[...])
    e_o[...] = _dot(h, w2[...]) + b2[...]


def _edge_net_body(ga, gb, e, wc, b1, w2, b2, e_o):
    h = jax.nn.gelu(_unpack_sum(ga[...], gb[...])
                    + _dot(e[...], wc[...]) + b1[...])
    e_o[...] = _dot(h, w2[...]) + b2[...] + e[...]


_RN = 1000
_RE = 2000


def _bs_rows(r):
    return pl.BlockSpec((r, D), lambda i: (i, 0))


_BS_W = pl.BlockSpec((D, D), lambda i: (0, 0))
_BS_B = pl.BlockSpec((1, D), lambda i: (0, 0))


def _tc_node_encoder(x, w1, b1, w2, b2, wea, web):
    return pl.pallas_call(
        _node_enc_body,
        grid=(N // _RN,),
        in_specs=[_bs_rows(_RN), _BS_W, _BS_B, _BS_W, _BS_B, _BS_W, _BS_W],
        out_specs=[_bs_rows(_RN)] * 3,
        out_shape=[jax.ShapeDtypeStruct((N, D), jnp.float32),
                   jax.ShapeDtypeStruct((N, D), jnp.bfloat16),
                   jax.ShapeDtypeStruct((N, D), jnp.bfloat16)],
    )(x, w1, b1, w2, b2, wea, web)


def _tc_node_net(x, m, w1a, w1b, b1, w2, b2, wea, web):
    return pl.pallas_call(
        _node_net_body,
        grid=(N // _RN,),
        in_specs=[_bs_rows(_RN), _bs_rows(_RN), _BS_W, _BS_W, _BS_B, _BS_W,
                  _BS_B, _BS_W, _BS_W],
        out_specs=[_bs_rows(_RN)] * 3,
        out_shape=[jax.ShapeDtypeStruct((N, D), jnp.float32),
                   jax.ShapeDtypeStruct((N, D), jnp.bfloat16),
                   jax.ShapeDtypeStruct((N, D), jnp.bfloat16)],
    )(x, m, w1a, w1b, b1, w2, b2, wea, web)


def _bs_pack(r):
    return pl.BlockSpec((r, D // 2), lambda i: (i, 0))


def _tc_edge_encoder(ga, gb, b1, w2, b2):
    return pl.pallas_call(
        _edge_enc_body,
        grid=(E // _RE,),
        in_specs=[_bs_pack(_RE), _bs_pack(_RE), _BS_B, _BS_W, _BS_B],
        out_specs=_bs_rows(_RE),
        out_shape=jax.ShapeDtypeStruct((E, D), jnp.float32),
    )(ga, gb, b1, w2, b2)


def _tc_edge_net(ga, gb, e, wc, b1, w2, b2):
    return pl.pallas_call(
        _edge_net_body,
        grid=(E // _RE,),
        in_specs=[_bs_pack(_RE), _bs_pack(_RE), _bs_rows(_RE), _BS_W, _BS_B,
                  _BS_W, _BS_B],
        out_specs=_bs_rows(_RE),
        out_shape=jax.ShapeDtypeStruct((E, D), jnp.float32),
    )(ga, gb, e, wc, b1, w2, b2)


GFULL = EPW // GC
GTAIL = EPW - GFULL * GC
GPAIRS = (GFULL - 2) // 2


def _gather2_body(a_hbm, b_hbm, src_hbm, dst_hbm, ga_hbm, gb_hbm,
                  idxa, idxb, ra0, rb0, ra1, rb1, sg0, sg1, sw0, sw1):
    c = lax.axis_index("c")
    s = lax.axis_index("s")
    wid = s * NC + c
    w0 = wid * EPW

    pltpu.sync_copy(src_hbm.at[pl.ds(w0, EPW)], idxa)
    pltpu.sync_copy(dst_hbm.at[pl.ds(w0, EPW)], idxb)

    def fire_g(j, ra, rb, sg):
        off = j * GC
        pltpu.async_copy(a_hbm.at[idxa.at[pl.ds(off, GC)]], ra, sg)
        pltpu.async_copy(b_hbm.at[idxb.at[pl.ds(off, GC)]], rb, sg)

    def wait_g(ra, rb, sg):
        pltpu.make_async_copy(a_hbm.at[pl.ds(0, GC)], ra, sg).wait()
        pltpu.make_async_copy(b_hbm.at[pl.ds(0, GC)], rb, sg).wait()

    def fire_w(j, ra, rb, sw):
        base = w0 + j * GC
        pltpu.async_copy(ra, ga_hbm.at[pl.ds(base, GC)], sw)
        pltpu.async_copy(rb, gb_hbm.at[pl.ds(base, GC)], sw)

    def wait_w(ra, rb, sw):
        pltpu.make_async_copy(ra, ga_hbm.at[pl.ds(0, GC)], sw).wait()
        pltpu.make_async_copy(rb, gb_hbm.at[pl.ds(0, GC)], sw).wait()

    fire_g(0, ra0, rb0, sg0)
    fire_g(1, ra1, rb1, sg1)
    wait_g(ra0, rb0, sg0)
    fire_w(0, ra0, rb0, sw0)
    wait_g(ra1, rb1, sg1)
    fire_w(1, ra1, rb1, sw1)

    def loop(t, carry):
        j0 = 2 + 2 * t
        wait_w(ra0, rb0, sw0)
        fire_g(j0, ra0, rb0, sg0)
        wait_w(ra1, rb1, sw1)
        fire_g(j0 + 1, ra1, rb1, sg1)
        wait_g(ra0, rb0, sg0)
        fire_w(j0, ra0, rb0, sw0)
        wait_g(ra1, rb1, sg1)
        fire_w(j0 + 1, ra1, rb1, sw1)
        return carry

    lax.fori_loop(0, GPAIRS, loop, 0)
    wait_w(ra0, rb0, sw0)
    wait_w(ra1, rb1, sw1)

    toff = GFULL * GC
    tbase = w0 + toff
    pltpu.async_copy(a_hbm.at[idxa.at[pl.ds(toff, GTAIL)]],
                     ra0.at[pl.ds(0, GTAIL)], sg0)
    pltpu.async_copy(b_hbm.at[idxb.at[pl.ds(toff, GTAIL)]],
                     rb0.at[pl.ds(0, GTAIL)], sg0)
    pltpu.make_async_copy(a_hbm.at[pl.ds(0, GTAIL)],
                          ra0.at[pl.ds(0, GTAIL)], sg0).wait()
    pltpu.make_async_copy(b_hbm.at[pl.ds(0, GTAIL)],
                          rb0.at[pl.ds(0, GTAIL)], sg0).wait()
    pltpu.sync_copy(ra0.at[pl.ds(0, GTAIL)], ga_hbm.at[pl.ds(tbase, GTAIL)])
    pltpu.sync_copy(rb0.at[pl.ds(0, GTAIL)], gb_hbm.at[pl.ds(tbase, GTAIL)])


def _sc_gather2(a, b, src, dst):
    f = pl.kernel(
        _gather2_body,
        out_type=[jax.ShapeDtypeStruct((E, D // 2), jnp.int32)] * 2,
        mesh=plsc.VectorSubcoreMesh(core_axis_name="c", subcore_axis_name="s"),
        scratch_types=[
            pltpu.VMEM((EPW,), jnp.int32),
            pltpu.VMEM((EPW,), jnp.int32),
            pltpu.VMEM((GC, D // 2), jnp.int32),
            pltpu.VMEM((GC, D // 2), jnp.int32),
            pltpu.VMEM((GC, D // 2), jnp.int32),
            pltpu.VMEM((GC, D // 2), jnp.int32),
            pltpu.SemaphoreType.DMA,
            pltpu.SemaphoreType.DMA,
            pltpu.SemaphoreType.DMA,
            pltpu.SemaphoreType.DMA,
        ],
    )
    return f(a, b, src, dst)


BASE_CH_S = (E // SC) // NS
EXTRA_S = (E // SC) - BASE_CH_S * NS


def _segsum_body(e_hbm, dst_hbm, zeros_hbm, msg_hbm,
                 idx0, rows0, idx1, rows1, sl0, sl1, ss0, ss1, acc):
    c = lax.axis_index("c")
    s = lax.axis_index("s")

    @pl.when(s < NS - 1)
    def _():
        pltpu.sync_copy(zeros_hbm.at[pl.ds(0, RPS)],
                        acc.at[pl.ds(s * RPS, RPS)])

    @pl.when(s == NS - 1)
    def _():
        pltpu.sync_copy(zeros_hbm, acc.at[pl.ds((NS - 1) * RPS, RPS_LAST)])

    plsc.subcore_barrier()

    def fire_l(j, idx, rows, sl):
        base = (s + NS * j) * SC
        pltpu.async_copy(dst_hbm.at[pl.ds(base, SC)], idx, sl)
        pltpu.async_copy(e_hbm.at[pl.ds(base, SC), pl.ds(c * CH, CH)],
                         rows, sl)

    def wait_l(idx, rows, sl):
        pltpu.make_async_copy(dst_hbm.at[pl.ds(0, SC)], idx, sl).wait()
        pltpu.make_async_copy(e_hbm.at[pl.ds(0, SC), pl.ds(0, CH)],
                              rows, sl).wait()

    def fire_s(idx, rows, ss):
        pltpu.async_copy(rows, acc.at[idx], ss, add=True)

    def wait_s(idx, rows, ss):
        pltpu.make_async_copy(rows, acc.at[idx], ss).wait()

    fire_l(0, idx0, rows0, sl0)
    fire_l(1, idx1, rows1, sl1)
    wait_l(idx0, rows0, sl0)
    fire_s(idx0, rows0, ss0)
    wait_l(idx1, rows1, sl1)
    fire_s(idx1, rows1, ss1)

    def loop(t, carry):
        j0 = 2 + 2 * t
        wait_s(idx0, rows0, ss0)
        fire_l(j0, idx0, rows0, sl0)
        wait_s(idx1, rows1, ss1)
        fire_l(j0 + 1, idx1, rows1, sl1)
        wait_l(idx0, rows0, sl0)
        fire_s(idx0, rows0, ss0)
        wait_l(idx1, rows1, sl1)
        fire_s(idx1, rows1, ss1)
        return carry

    lax.fori_loop(0, (BASE_CH_S - 2) // 2, loop, 0)
    wait_s(idx0, rows0, ss0)
    wait_s(idx1, rows1, ss1)

    @pl.when(s < EXTRA_S)
    def _():
        base = (NS * BASE_CH_S + s) * SC
        pltpu.sync_copy(dst_hbm.at[pl.ds(base, SC)], idx0)
        pltpu.sync_copy(e_hbm.at[pl.ds(base, SC), pl.ds(c * CH, CH)], rows0)
        pltpu.sync_copy(rows0, acc.at[idx0], add=True)

    plsc.subcore_barrier()

    @pl.when(s < NS - 1)
    def _():
        pltpu.sync_copy(acc.at[pl.ds(s * RPS, RPS)],
                        msg_hbm.at[pl.ds(s * RPS, RPS), pl.ds(c * CH, CH)])

    @pl.when(s == NS - 1)
    def _():
        pltpu.sync_copy(
            acc.at[pl.ds((NS - 1) * RPS, RPS_LAST)],
            msg_hbm.at[pl.ds((NS - 1) * RPS, RPS_LAST), pl.ds(c * CH, CH)])


def _sc_segsum(edges, dst, zeros):
    f = pl.kernel(
        _segsum_body,
        out_type=jax.ShapeDtypeStruct((N, D), jnp.float32),
        mesh=plsc.VectorSubcoreMesh(core_axis_name="c", subcore_axis_name="s"),
        scratch_types=[
            pltpu.VMEM((SC,), jnp.int32),
            pltpu.VMEM((SC, CH), jnp.float32),
            pltpu.VMEM((SC,), jnp.int32),
            pltpu.VMEM((SC, CH), jnp.float32),
            pltpu.SemaphoreType.DMA,
            pltpu.SemaphoreType.DMA,
            pltpu.SemaphoreType.DMA,
            pltpu.SemaphoreType.DMA,
            pltpu.VMEM_SHARED((N, CH), jnp.float32),
        ],
    )
    return f(edges, dst, zeros)


_DEINT = np.array(list(range(0, D, 2)) + list(range(1, D, 2)), np.int32)


def _pack(x_bf):
    return jax.lax.bitcast_convert_type(
        x_bf.reshape(-1, D // 2, 2), jnp.int32)


def kernel(node_attr, graph, params):
    src = graph[0].astype(jnp.int32)
    dst = graph[1].astype(jnp.int32)

    enc = params["node_encoder"]
    ee = params["edge_encoder"]
    w1n, b1n = enc[0]["W"], enc[0]["b"].reshape(1, D)
    w2n, b2n = enc[1]["W"], enc[1]["b"].reshape(1, D)
    we1 = ee[0]["W"]
    be1 = ee[0]["b"][_DEINT].reshape(1, D)
    we2, be2 = ee[1]["W"][_DEINT, :], ee[1]["b"].reshape(1, D)

    nodes, a, b = _tc_node_encoder(node_attr, w1n, b1n, w2n, b2n,
                                   we1[:D], we1[D:])
    ga, gb = _sc_gather2(_pack(a), _pack(b), src, dst)
    edges = _tc_edge_encoder(ga, gb, be1, we2, be2)

    zeros = jnp.zeros((RPS_LAST, CH), jnp.float32)
    for cell in params["cells"]:
        nw, ew = cell["node_network"], cell["edge_network"]
        wn1, bn1 = nw[0]["W"], nw[0]["b"].reshape(1, D)
        wn2, bn2 = nw[1]["W"], nw[1]["b"].reshape(1, D)
        wc1 = ew[0]["W"]
        bc1 = ew[0]["b"][_DEINT].reshape(1, D)
        wcc = wc1[2 * D:][:, _DEINT]
        wc2, bc2 = ew[1]["W"][_DEINT, :], ew[1]["b"].reshape(1, D)

        msg = _sc_segsum(edges, dst, zeros)
        nodes, a, b = _tc_node_net(nodes, msg, wn1[:D], wn1[D:], bn1,
                                   wn2, bn2, wc1[:D], wc1[D:2 * D])
        ga, gb = _sc_gather2(_pack(a), _pack(b), src, dst)
        edges = _tc_edge_net(ga, gb, edges, wcc, bc1, wc2, bc2)

    return (nodes, edges)


# speedup vs baseline: 1.9962x; 1.4797x over previous
import functools

import jax
import jax.numpy as jnp
import numpy as np
from jax import lax
from jax.experimental import pallas as pl
from jax.experimental.pallas import tpu as pltpu
from jax.experimental.pallas import tpu_sc as plsc

N = 10000
E = 160000
D = 256

NC = 2
NS = 16
NW = NC * NS

SC = 128
GC = 112
EPW = E // NW

RPS = 624
RPS_LAST = N - RPS * (NS - 1)
CH = D // NC


def _dot(a, b):
    return jnp.dot(a, b, preferred_element_type=jnp.float32)


_HI_MASK = np.int32(-65536)


def _pack_tc(x):
    bl = pltpu.bitcast(x[:, :D // 2], jnp.int32)
    bh = pltpu.bitcast(x[:, D // 2:], jnp.int32)
    tl = bl + 0x7FFF + ((bl >> 16) & 1)
    th = bh + 0x7FFF + ((bh >> 16) & 1)
    return ((tl >> 16) & 0xFFFF) | (th & _HI_MASK)


def _node_enc_body(x, w1, b1, w2, b2, wea, web, nd_o, a_o, b_o):
    h = jax.nn.gelu(_dot(x[...], w1[...]) + b1[...])
    nd = _dot(h, w2[...]) + b2[...]
    nd_o[...] = nd
    a_o[...] = _pack_tc(_dot(nd, wea[...]))
    b_o[...] = _pack_tc(_dot(nd, web[...]))


def _node_net_body(x, m, w1a, w1b, b1, w2, b2, wea, web, nd_o, a_o, b_o):
    h = jax.nn.gelu(_dot(x[...], w1a[...]) + _dot(m[...], w1b[...]) + b1[...])
    nd = _dot(h, w2[...]) + b2[...] + x[...]
    nd_o[...] = nd
    a_o[...] = _pack_tc(_dot(nd, wea[...]))
    b_o[...] = _pack_tc(_dot(nd, web[...]))


def _unpack_sum(a, b):
    lo = (pltpu.bitcast(a << 16, jnp.float32)
          + pltpu.bitcast(b << 16, jnp.float32))
    hi = (pltpu.bitcast(a & _HI_MASK, jnp.float32)
          + pltpu.bitcast(b & _HI_MASK, jnp.float32))
    return jnp.concatenate([lo, hi], axis=1)


def _edge_enc_body(ga, gb, b1, w2, b2, e_o):
    h = jax.nn.gelu(_unpack_sum(ga[...], gb[...]) + b1[...])
    e_o[...] = _dot(h, w2[...]) + b2[...]


def _edge_net_body(ga, gb, e, wc, b1, w2, b2, e_o):
    h = jax.nn.gelu(_unpack_sum(ga[...], gb[...])
                    + _dot(e[...], wc[...]) + b1[...])
    e_o[...] = _dot(h, w2[...]) + b2[...] + e[...]


_RN = 1000
_RE = 2000


def _bs_rows(r):
    return pl.BlockSpec((r, D), lambda i: (i, 0))


def _bs_pack(r):
    return pl.BlockSpec((r, D // 2), lambda i: (i, 0))


_BS_W = pl.BlockSpec((D, D), lambda i: (0, 0))
_BS_B = pl.BlockSpec((1, D), lambda i: (0, 0))


def _tc_node_encoder(x, w1, b1, w2, b2, wea, web):
    return pl.pallas_call(
        _node_enc_body,
        grid=(N // _RN,),
        in_specs=[_bs_rows(_RN), _BS_W, _BS_B, _BS_W, _BS_B, _BS_W, _BS_W],
        out_specs=[_bs_rows(_RN), _bs_pack(_RN), _bs_pack(_RN)],
        out_shape=[jax.ShapeDtypeStruct((N, D), jnp.float32),
                   jax.ShapeDtypeStruct((N, D // 2), jnp.int32),
                   jax.ShapeDtypeStruct((N, D // 2), jnp.int32)],
    )(x, w1, b1, w2, b2, wea, web)


def _tc_node_net(x, m, w1a, w1b, b1, w2, b2, wea, web):
    return pl.pallas_call(
        _node_net_body,
        grid=(N // _RN,),
        in_specs=[_bs_rows(_RN), _bs_rows(_RN), _BS_W, _BS_W, _BS_B, _BS_W,
                  _BS_B, _BS_W, _BS_W],
        out_specs=[_bs_rows(_RN), _bs_pack(_RN), _bs_pack(_RN)],
        out_shape=[jax.ShapeDtypeStruct((N, D), jnp.float32),
                   jax.ShapeDtypeStruct((N, D // 2), jnp.int32),
                   jax.ShapeDtypeStruct((N, D // 2), jnp.int32)],
    )(x, m, w1a, w1b, b1, w2, b2, wea, web)


def _tc_edge_encoder(ga, gb, b1, w2, b2):
    return pl.pallas_call(
        _edge_enc_body,
        grid=(E // _RE,),
        in_specs=[_bs_pack(_RE), _bs_pack(_RE), _BS_B, _BS_W, _BS_B],
        out_specs=_bs_rows(_RE),
        out_shape=jax.ShapeDtypeStruct((E, D), jnp.float32),
    )(ga, gb, b1, w2, b2)


def _tc_edge_net(ga, gb, e, wc, b1, w2, b2):
    return pl.pallas_call(
        _edge_net_body,
        grid=(E // _RE,),
        in_specs=[_bs_pack(_RE), _bs_pack(_RE), _bs_rows(_RE), _BS_W, _BS_B,
                  _BS_W, _BS_B],
        out_specs=_bs_rows(_RE),
        out_shape=jax.ShapeDtypeStruct((E, D), jnp.float32),
    )(ga, gb, e, wc, b1, w2, b2)


GFULL = EPW // GC
GTAIL = EPW - GFULL * GC
GPAIRS = (GFULL - 2) // 2


def _gather2_body(a_hbm, b_hbm, src_hbm, dst_hbm, ga_hbm, gb_hbm,
                  idxa, idxb, ra0, rb0, ra1, rb1, sg0, sg1, sw0, sw1):
    c = lax.axis_index("c")
    s = lax.axis_index("s")
    wid = s * NC + c
    w0 = wid * EPW

    pltpu.sync_copy(src_hbm.at[pl.ds(w0, EPW)], idxa)
    pltpu.sync_copy(dst_hbm.at[pl.ds(w0, EPW)], idxb)

    def fire_g(j, ra, rb, sg):
        off = j * GC
        pltpu.async_copy(a_hbm.at[idxa.at[pl.ds(off, GC)]], ra, sg)
        pltpu.async_copy(b_hbm.at[idxb.at[pl.ds(off, GC)]], rb, sg)

    def wait_g(ra, rb, sg):
        pltpu.make_async_copy(a_hbm.at[pl.ds(0, GC)], ra, sg).wait()
        pltpu.make_async_copy(b_hbm.at[pl.ds(0, GC)], rb, sg).wait()

    def fire_w(j, ra, rb, sw):
        base = w0 + j * GC
        pltpu.async_copy(ra, ga_hbm.at[pl.ds(base, GC)], sw)
        pltpu.async_copy(rb, gb_hbm.at[pl.ds(base, GC)], sw)

    def wait_w(ra, rb, sw):
        pltpu.make_async_copy(ra, ga_hbm.at[pl.ds(0, GC)], sw).wait()
        pltpu.make_async_copy(rb, gb_hbm.at[pl.ds(0, GC)], sw).wait()

    fire_g(0, ra0, rb0, sg0)
    fire_g(1, ra1, rb1, sg1)
    wait_g(ra0, rb0, sg0)
    fire_w(0, ra0, rb0, sw0)
    wait_g(ra1, rb1, sg1)
    fire_w(1, ra1, rb1, sw1)

    def loop(t, carry):
        j0 = 2 + 2 * t
        wait_w(ra0, rb0, sw0)
        fire_g(j0, ra0, rb0, sg0)
        wait_w(ra1, rb1, sw1)
        fire_g(j0 + 1, ra1, rb1, sg1)
        wait_g(ra0, rb0, sg0)
        fire_w(j0, ra0, rb0, sw0)
        wait_g(ra1, rb1, sg1)
        fire_w(j0 + 1, ra1, rb1, sw1)
        return carry

    lax.fori_loop(0, GPAIRS, loop, 0)
    wait_w(ra0, rb0, sw0)
    wait_w(ra1, rb1, sw1)

    toff = GFULL * GC
    tbase = w0 + toff
    pltpu.async_copy(a_hbm.at[idxa.at[pl.ds(toff, GTAIL)]],
                     ra0.at[pl.ds(0, GTAIL)], sg0)
    pltpu.async_copy(b_hbm.at[idxb.at[pl.ds(toff, GTAIL)]],
                     rb0.at[pl.ds(0, GTAIL)], sg0)
    pltpu.make_async_copy(a_hbm.at[pl.ds(0, GTAIL)],
                          ra0.at[pl.ds(0, GTAIL)], sg0).wait()
    pltpu.make_async_copy(b_hbm.at[pl.ds(0, GTAIL)],
                          rb0.at[pl.ds(0, GTAIL)], sg0).wait()
    pltpu.sync_copy(ra0.at[pl.ds(0, GTAIL)], ga_hbm.at[pl.ds(tbase, GTAIL)])
    pltpu.sync_copy(rb0.at[pl.ds(0, GTAIL)], gb_hbm.at[pl.ds(tbase, GTAIL)])


def _sc_gather2(a, b, src, dst):
    f = pl.kernel(
        _gather2_body,
        out_type=[jax.ShapeDtypeStruct((E, D // 2), jnp.int32)] * 2,
        mesh=plsc.VectorSubcoreMesh(core_axis_name="c", subcore_axis_name="s"),
        scratch_types=[
            pltpu.VMEM((EPW,), jnp.int32),
            pltpu.VMEM((EPW,), jnp.int32),
            pltpu.VMEM((GC, D // 2), jnp.int32),
            pltpu.VMEM((GC, D // 2), jnp.int32),
            pltpu.VMEM((GC, D // 2), jnp.int32),
            pltpu.VMEM((GC, D // 2), jnp.int32),
            pltpu.SemaphoreType.DMA,
            pltpu.SemaphoreType.DMA,
            pltpu.SemaphoreType.DMA,
            pltpu.SemaphoreType.DMA,
        ],
    )
    return f(a, b, src, dst)


BASE_CH_S = (E // SC) // NS
EXTRA_S = (E // SC) - BASE_CH_S * NS


def _segsum_body(e_hbm, dst_hbm, zeros_hbm, msg_hbm,
                 idx0, rows0, idx1, rows1, sl0, sl1, ss0, ss1, acc):
    c = lax.axis_index("c")
    s = lax.axis_index("s")

    @pl.when(s < NS - 1)
    def _():
        pltpu.sync_copy(zeros_hbm.at[pl.ds(0, RPS)],
                        acc.at[pl.ds(s * RPS, RPS)])

    @pl.when(s == NS - 1)
    def _():
        pltpu.sync_copy(zeros_hbm, acc.at[pl.ds((NS - 1) * RPS, RPS_LAST)])

    plsc.subcore_barrier()

    def fire_l(j, idx, rows, sl):
        base = (s + NS * j) * SC
        pltpu.async_copy(dst_hbm.at[pl.ds(base, SC)], idx, sl)
        pltpu.async_copy(e_hbm.at[pl.ds(base, SC), pl.ds(c * CH, CH)],
                         rows, sl)

    def wait_l(idx, rows, sl):
        pltpu.make_async_copy(dst_hbm.at[pl.ds(0, SC)], idx, sl).wait()
        pltpu.make_async_copy(e_hbm.at[pl.ds(0, SC), pl.ds(0, CH)],
                              rows, sl).wait()

    def fire_s(idx, rows, ss):
        pltpu.async_copy(rows, acc.at[idx], ss, add=True)

    def wait_s(idx, rows, ss):
        pltpu.make_async_copy(rows, acc.at[idx], ss).wait()

    fire_l(0, idx0, rows0, sl0)
    fire_l(1, idx1, rows1, sl1)
    wait_l(idx0, rows0, sl0)
    fire_s(idx0, rows0, ss0)
    wait_l(idx1, rows1, sl1)
    fire_s(idx1, rows1, ss1)

    def loop(t, carry):
        j0 = 2 + 2 * t
        wait_s(idx0, rows0, ss0)
        fire_l(j0, idx0, rows0, sl0)
        wait_s(idx1, rows1, ss1)
        fire_l(j0 + 1, idx1, rows1, sl1)
        wait_l(idx0, rows0, sl0)
        fire_s(idx0, rows0, ss0)
        wait_l(idx1, rows1, sl1)
        fire_s(idx1, rows1, ss1)
        return carry

    lax.fori_loop(0, (BASE_CH_S - 2) // 2, loop, 0)
    wait_s(idx0, rows0, ss0)
    wait_s(idx1, rows1, ss1)

    @pl.when(s < EXTRA_S)
    def _():
        base = (NS * BASE_CH_S + s) * SC
        pltpu.sync_copy(dst_hbm.at[pl.ds(base, SC)], idx0)
        pltpu.sync_copy(e_hbm.at[pl.ds(base, SC), pl.ds(c * CH, CH)], rows0)
        pltpu.sync_copy(rows0, acc.at[idx0], add=True)

    plsc.subcore_barrier()

    @pl.when(s < NS - 1)
    def _():
        pltpu.sync_copy(acc.at[pl.ds(s * RPS, RPS)],
                        msg_hbm.at[pl.ds(s * RPS, RPS), pl.ds(c * CH, CH)])

    @pl.when(s == NS - 1)
    def _():
        pltpu.sync_copy(
            acc.at[pl.ds((NS - 1) * RPS, RPS_LAST)],
            msg_hbm.at[pl.ds((NS - 1) * RPS, RPS_LAST), pl.ds(c * CH, CH)])


def _sc_segsum(edges, dst, zeros):
    f = pl.kernel(
        _segsum_body,
        out_type=jax.ShapeDtypeStruct((N, D), jnp.float32),
        mesh=plsc.VectorSubcoreMesh(core_axis_name="c", subcore_axis_name="s"),
        scratch_types=[
            pltpu.VMEM((SC,), jnp.int32),
            pltpu.VMEM((SC, CH), jnp.float32),
            pltpu.VMEM((SC,), jnp.int32),
            pltpu.VMEM((SC, CH), jnp.float32),
            pltpu.SemaphoreType.DMA,
            pltpu.SemaphoreType.DMA,
            pltpu.SemaphoreType.DMA,
            pltpu.SemaphoreType.DMA,
            pltpu.VMEM_SHARED((N, CH), jnp.float32),
        ],
    )
    return f(edges, dst, zeros)


def kernel(node_attr, graph, params):
    src = graph[0].astype(jnp.int32)
    dst = graph[1].astype(jnp.int32)

    enc = params["node_encoder"]
    ee = params["edge_encoder"]
    w1n, b1n = enc[0]["W"], enc[0]["b"].reshape(1, D)
    w2n, b2n = enc[1]["W"], enc[1]["b"].reshape(1, D)
    we1, be1 = ee[0]["W"], ee[0]["b"].reshape(1, D)
    we2, be2 = ee[1]["W"], ee[1]["b"].reshape(1, D)

    nodes, a, b = _tc_node_encoder(node_attr, w1n, b1n, w2n, b2n,
                                   we1[:D], we1[D:])
    ga, gb = _sc_gather2(a, b, src, dst)
    edges = _tc_edge_encoder(ga, gb, be1, we2, be2)

    zeros = jnp.zeros((RPS_LAST, CH), jnp.float32)
    for cell in params["cells"]:
        nw, ew = cell["node_network"], cell["edge_network"]
        wn1, bn1 = nw[0]["W"], nw[0]["b"].reshape(1, D)
        wn2, bn2 = nw[1]["W"], nw[1]["b"].reshape(1, D)
        wc1, bc1 = ew[0]["W"], ew[0]["b"].reshape(1, D)
        wc2, bc2 = ew[1]["W"], ew[1]["b"].reshape(1, D)

        msg = _sc_segsum(edges, dst, zeros)
        nodes, a, b = _tc_node_net(nodes, msg, wn1[:D], wn1[D:], bn1,
                                   wn2, bn2, wc1[:D], wc1[D:2 * D])
        ga, gb = _sc_gather2(a, b, src, dst)
        edges = _tc_edge_net(ga, gb, edges, wc1[2 * D:], bc1, wc2, bc2)

    return (nodes, edges)
